# baseline (speedup 1.0000x reference)
import functools

import jax
import jax.numpy as jnp
from jax import lax
from jax.experimental import pallas as pl
from jax.experimental.pallas import tpu as pltpu
from jax.experimental.pallas import tpu_sc as plsc

N_NODES = 10000
NPAD = 10240
ROWS_PER_TILE = NPAD // 16
E = 320000
K = 128
NCHUNKS = E // K
NW = 32
DEG_LANES = 16


def _mesh():
    return plsc.VectorSubcoreMesh(core_axis_name="c", subcore_axis_name="s")


_CPT = NCHUNKS // NW
_REM = NCHUNKS - _CPT * NW
_NT = _CPT // 3


def _deg_body(dst2_hbm, zeros_hbm, out_hbm, didx0, didx1, didx2, obuf, acc,
              dsem0, dsem1, dsem2, ssem0, ssem1, ssem2):
    c = lax.axis_index("c")
    s = lax.axis_index("s")
    wid = s * 2 + c
    c0 = wid * _CPT

    def fill(i, carry):
        obuf[i, :] = jnp.full((16,), 1.0, jnp.float32)
        return carry

    lax.fori_loop(0, K, fill, 0)
    row0 = s * ROWS_PER_TILE
    pltpu.sync_copy(zeros_hbm.at[pl.ds(row0, ROWS_PER_TILE)],
                    acc.at[pl.ds(row0, ROWS_PER_TILE)])
    plsc.subcore_barrier()

    pltpu.async_copy(dst2_hbm.at[c0], didx0, dsem0)
    pltpu.async_copy(dst2_hbm.at[c0 + 1], didx1, dsem1)
    pltpu.async_copy(dst2_hbm.at[c0 + 2], didx2, dsem2)

    slots = [(didx0, dsem0, ssem0), (didx1, dsem1, ssem1),
             (didx2, dsem2, ssem2)]

    def ring(t, carry):
        j0 = c0 + 3 * t
        for b, (didx, dsem, ssem) in enumerate(slots):
            pltpu.make_async_copy(dst2_hbm.at[j0 + b], didx, dsem).wait()
            pltpu.async_copy(obuf, acc.at[didx], ssem, add=True)

        @pl.when(t < _NT - 1)
        def _():
            for b, (didx, dsem, ssem) in enumerate(slots):
                pltpu.make_async_copy(obuf, acc.at[didx], ssem).wait()
                pltpu.async_copy(dst2_hbm.at[j0 + b + 3], didx, dsem)

        return carry

    lax.fori_loop(0, _NT, ring, 0)
    pltpu.make_async_copy(obuf, acc.at[didx0], ssem0).wait()
    pltpu.make_async_copy(obuf, acc.at[didx1], ssem1).wait()
    pltpu.make_async_copy(obuf, acc.at[didx2], ssem2).wait()

    @pl.when(wid < _REM)
    def _():
        ct = _CPT * NW + wid
        pltpu.sync_copy(dst2_hbm.at[ct], didx0)
        pltpu.sync_copy(obuf, acc.at[didx0], add=True)

    plsc.subcore_barrier()
    pltpu.sync_copy(acc.at[pl.ds(row0, ROWS_PER_TILE)],
                    out_hbm.at[c].at[pl.ds(row0, ROWS_PER_TILE)])


def _degree_partials(dst2, zeros_rows):
    kern = pl.kernel(
        _deg_body,
        out_type=jax.ShapeDtypeStruct((2, NPAD, DEG_LANES), jnp.float32),
        mesh=_mesh(),
        compiler_params=pltpu.CompilerParams(use_tc_tiling_on_sc=False),
        scratch_types=[
            pltpu.VMEM((K,), jnp.int32),
            pltpu.VMEM((K,), jnp.int32),
            pltpu.VMEM((K,), jnp.int32),
            pltpu.VMEM((K, DEG_LANES), jnp.float32),
            pltpu.VMEM_SHARED((NPAD, DEG_LANES), jnp.float32),
            pltpu.SemaphoreType.DMA,
            pltpu.SemaphoreType.DMA,
            pltpu.SemaphoreType.DMA,
            pltpu.SemaphoreType.DMA,
            pltpu.SemaphoreType.DMA,
            pltpu.SemaphoreType.DMA,
        ],
    )
    return kern(dst2, zeros_rows)


def _make_agg(D):
    def body(y_hbm, src2_hbm, dst2_hbm, out_hbm,
             sidx_all, didx0, didx1, rows0, rows1,
             acc, isem, gsem0, gsem1, dsem0, dsem1, ssem0, ssem1):
        c = lax.axis_index("c")
        s = lax.axis_index("s")
        wid = s * 2 + c
        row0 = s * ROWS_PER_TILE
        c0 = wid * _CPT
        i1 = pltpu.async_copy(src2_hbm.at[pl.ds(c0, _CPT)], sidx_all, isem)
        pltpu.sync_copy(y_hbm.at[pl.ds(row0, ROWS_PER_TILE)],
                        acc.at[pl.ds(row0, ROWS_PER_TILE)])
        plsc.subcore_barrier()
        i1.wait()

        slots = [(didx0, dsem0, gsem0, ssem0, rows0),
                 (didx1, dsem1, gsem1, ssem1, rows1)]
        for b, (didx, dsem, gsem, ssem, rows) in enumerate(slots):
            pltpu.async_copy(dst2_hbm.at[c0 + b], didx, dsem)
            pltpu.async_copy(y_hbm.at[sidx_all.at[b]], rows, gsem)

        def ring(t, carry):
            j0 = c0 + 2 * t
            for b, (didx, dsem, gsem, ssem, rows) in enumerate(slots):
                pltpu.make_async_copy(y_hbm.at[sidx_all.at[0]], rows,
                                      gsem).wait()
                pltpu.make_async_copy(dst2_hbm.at[j0 + b], didx, dsem).wait()
                pltpu.async_copy(rows, acc.at[didx], ssem, add=True)

            @pl.when(t < _CPT // 2 - 1)
            def _():
                for b, (didx, dsem, gsem, ssem, rows) in enumerate(slots):
                    pltpu.make_async_copy(rows, acc.at[didx], ssem).wait()
                    pltpu.async_copy(dst2_hbm.at[j0 + b + 2], didx, dsem)
                    pltpu.async_copy(y_hbm.at[sidx_all.at[2 * t + b + 2]],
                                     rows, gsem)

            return carry

        lax.fori_loop(0, _CPT // 2, ring, 0)
        for b, (didx, dsem, gsem, ssem, rows) in enumerate(slots):
            pltpu.make_async_copy(rows, acc.at[didx], ssem).wait()

        @pl.when(wid < _REM)
        def _():
            ct = _CPT * NW + wid
            pltpu.sync_copy(src2_hbm.at[ct], didx0)
            pltpu.sync_copy(dst2_hbm.at[ct], didx1)
            pltpu.async_copy(y_hbm.at[didx0], rows0, gsem0).wait()
            pltpu.sync_copy(rows0, acc.at[didx1], add=True)

        plsc.subcore_barrier()
        pltpu.sync_copy(acc.at[pl.ds(row0, ROWS_PER_TILE)],
                        out_hbm.at[c].at[pl.ds(row0, ROWS_PER_TILE)])

    kern = pl.kernel(
        body,
        out_type=jax.ShapeDtypeStruct((2, NPAD, D), jnp.float32),
        mesh=_mesh(),
        compiler_params=pltpu.CompilerParams(use_tc_tiling_on_sc=False),
        scratch_types=[
            pltpu.VMEM((_CPT, K), jnp.int32),
            pltpu.VMEM((K,), jnp.int32),
            pltpu.VMEM((K,), jnp.int32),
            pltpu.VMEM((K, D), jnp.float32),
            pltpu.VMEM((K, D), jnp.float32),
            pltpu.VMEM_SHARED((NPAD, D), jnp.float32),
            pltpu.SemaphoreType.DMA,
            pltpu.SemaphoreType.DMA,
            pltpu.SemaphoreType.DMA,
            pltpu.SemaphoreType.DMA,
            pltpu.SemaphoreType.DMA,
            pltpu.SemaphoreType.DMA,
            pltpu.SemaphoreType.DMA,
        ],
    )
    return kern


_RB = 640


def _prep1_body(x_ref, w_ref, degp_ref, y_ref, dinv_ref):
    deg = degp_ref[0, :, 0:1] + degp_ref[1, :, 0:1] + 1.0
    dinv = lax.rsqrt(deg)
    xw = jnp.dot(x_ref[...], w_ref[...], preferred_element_type=jnp.float32)
    y_ref[...] = xw * dinv
    dinv_ref[...] = dinv


def _prep1(x_pad, w1, degp):
    grid = NPAD // _RB
    return pl.pallas_call(
        _prep1_body,
        grid=(grid,),
        in_specs=[
            pl.BlockSpec((_RB, 128), lambda i: (i, 0)),
            pl.BlockSpec((128, 128), lambda i: (0, 0)),
            pl.BlockSpec((2, _RB, DEG_LANES), lambda i: (0, i, 0)),
        ],
        out_specs=[
            pl.BlockSpec((_RB, 128), lambda i: (i, 0)),
            pl.BlockSpec((_RB, 1), lambda i: (i, 0)),
        ],
        out_shape=[
            jax.ShapeDtypeStruct((NPAD, 128), jnp.float32),
            jax.ShapeDtypeStruct((NPAD, 1), jnp.float32),
        ],
    )(x_pad, w1, degp)


def _mid_body(p0_ref, p1_ref, y_ref, dinv_ref, b_ref, w_ref, y2_ref):
    dinv = dinv_ref[...]
    h = dinv * (p0_ref[...] + p1_ref[...] - y_ref[...]) + b_ref[...]
    h = jnp.maximum(h, 0.0)
    y2_ref[...] = jnp.dot(h, w_ref[...], preferred_element_type=jnp.float32) * dinv


def _mid(p0, p1, y1, dinv, b1, w2):
    grid = NPAD // _RB
    return pl.pallas_call(
        _mid_body,
        grid=(grid,),
        in_specs=[
            pl.BlockSpec((_RB, 128), lambda i: (i, 0)),
            pl.BlockSpec((_RB, 128), lambda i: (i, 0)),
            pl.BlockSpec((_RB, 128), lambda i: (i, 0)),
            pl.BlockSpec((_RB, 1), lambda i: (i, 0)),
            pl.BlockSpec((1, 128), lambda i: (0, 0)),
            pl.BlockSpec((128, 64), lambda i: (0, 0)),
        ],
        out_specs=pl.BlockSpec((_RB, 64), lambda i: (i, 0)),
        out_shape=jax.ShapeDtypeStruct((NPAD, 64), jnp.float32),
    )(p0, p1, y1, dinv, b1, w2)


def _final_body(p0_ref, p1_ref, y_ref, dinv_ref, b_ref, o_ref):
    t = dinv_ref[...] * (p0_ref[...] + p1_ref[...] - y_ref[...]) + b_ref[...]
    m = jnp.max(t, axis=1, keepdims=True)
    e = t - m
    lse = jnp.log(jnp.sum(jnp.exp(e), axis=1, keepdims=True))
    o_ref[...] = e - lse


def _final(p0, p1, y2, dinv, b2):
    grid = NPAD // _RB
    return pl.pallas_call(
        _final_body,
        grid=(grid,),
        in_specs=[
            pl.BlockSpec((_RB, 64), lambda i: (i, 0)),
            pl.BlockSpec((_RB, 64), lambda i: (i, 0)),
            pl.BlockSpec((_RB, 64), lambda i: (i, 0)),
            pl.BlockSpec((_RB, 1), lambda i: (i, 0)),
            pl.BlockSpec((1, 64), lambda i: (0, 0)),
        ],
        out_specs=pl.BlockSpec((_RB, 64), lambda i: (i, 0)),
        out_shape=jax.ShapeDtypeStruct((NPAD, 64), jnp.float32),
    )(p0, p1, y2, dinv, b2)


@jax.jit
def kernel(x, edge_index, W1, b1, W2, b2):
    ei = edge_index.astype(jnp.int32)
    src = ei[0]
    dst = ei[1]
    src2 = src.reshape(NCHUNKS, K)
    dst2 = dst.reshape(NCHUNKS, K)
    x_pad = jnp.pad(x, ((0, NPAD - N_NODES), (0, 0)))
    zeros_rows = jnp.zeros((NPAD, DEG_LANES), jnp.float32)

    degp = _degree_partials(dst2, zeros_rows)
    y1, dinv = _prep1(x_pad, W1, degp)

    agg128 = _make_agg(128)
    p1 = agg128(y1, src2, dst2)
    y2 = _mid(p1[0], p1[1], y1, dinv, b1.reshape(1, 128), W2)

    agg64 = _make_agg(64)
    p2 = agg64(y2, src2, dst2)
    out = _final(p2[0], p2[1], y2, dinv, b2.reshape(1, 64))
    return out[:N_NODES]


# speedup vs baseline: 1.1199x; 1.1199x over previous
import functools

import jax
import jax.numpy as jnp
from jax import lax
from jax.experimental import pallas as pl
from jax.experimental.pallas import tpu as pltpu
from jax.experimental.pallas import tpu_sc as plsc

N_NODES = 10000
NPAD = 10240
ROWS_PER_TILE = NPAD // 16
E = 320000
K = 128
NCHUNKS = E // K
NW = 32
DEG_LANES = 16


def _mesh():
    return plsc.VectorSubcoreMesh(core_axis_name="c", subcore_axis_name="s")


_CPT = NCHUNKS // NW
_REM = NCHUNKS - _CPT * NW
_NT = _CPT // 3


def _deg_body(ei3_hbm, zeros_hbm, out0_hbm, out1_hbm, didx0, didx1, didx2,
              obuf, acc, dsem0, dsem1, dsem2, ssem0, ssem1, ssem2):
    dst2_hbm = ei3_hbm.at[1]
    c = lax.axis_index("c")
    s = lax.axis_index("s")
    wid = s * 2 + c
    c0 = wid * _CPT

    def fill(i, carry):
        obuf[i, :] = jnp.full((16,), 1.0, jnp.float32)
        return carry

    lax.fori_loop(0, K, fill, 0)
    row0 = s * ROWS_PER_TILE
    pltpu.sync_copy(zeros_hbm.at[pl.ds(row0, ROWS_PER_TILE)],
                    acc.at[pl.ds(row0, ROWS_PER_TILE)])
    plsc.subcore_barrier()

    pltpu.async_copy(dst2_hbm.at[c0], didx0, dsem0)
    pltpu.async_copy(dst2_hbm.at[c0 + 1], didx1, dsem1)
    pltpu.async_copy(dst2_hbm.at[c0 + 2], didx2, dsem2)

    slots = [(didx0, dsem0, ssem0), (didx1, dsem1, ssem1),
             (didx2, dsem2, ssem2)]

    def ring(t, carry):
        j0 = c0 + 3 * t
        for b, (didx, dsem, ssem) in enumerate(slots):
            pltpu.make_async_copy(dst2_hbm.at[j0 + b], didx, dsem).wait()
            pltpu.async_copy(obuf, acc.at[didx], ssem, add=True)

        @pl.when(t < _NT - 1)
        def _():
            for b, (didx, dsem, ssem) in enumerate(slots):
                pltpu.make_async_copy(obuf, acc.at[didx], ssem).wait()
                pltpu.async_copy(dst2_hbm.at[j0 + b + 3], didx, dsem)

        return carry

    lax.fori_loop(0, _NT, ring, 0)
    pltpu.make_async_copy(obuf, acc.at[didx0], ssem0).wait()
    pltpu.make_async_copy(obuf, acc.at[didx1], ssem1).wait()
    pltpu.make_async_copy(obuf, acc.at[didx2], ssem2).wait()

    @pl.when(wid < _REM)
    def _():
        ct = _CPT * NW + wid
        pltpu.sync_copy(dst2_hbm.at[ct], didx0)
        pltpu.sync_copy(obuf, acc.at[didx0], add=True)

    plsc.subcore_barrier()

    @pl.when(c == 0)
    def _():
        pltpu.sync_copy(acc.at[pl.ds(row0, ROWS_PER_TILE)],
                        out0_hbm.at[pl.ds(row0, ROWS_PER_TILE)])

    @pl.when(c == 1)
    def _():
        pltpu.sync_copy(acc.at[pl.ds(row0, ROWS_PER_TILE)],
                        out1_hbm.at[pl.ds(row0, ROWS_PER_TILE)])


def _degree_partials(ei3, zeros_rows):
    kern = pl.kernel(
        _deg_body,
        out_type=(jax.ShapeDtypeStruct((NPAD, DEG_LANES), jnp.float32),
                  jax.ShapeDtypeStruct((NPAD, DEG_LANES), jnp.float32)),
        mesh=_mesh(),
        compiler_params=pltpu.CompilerParams(use_tc_tiling_on_sc=False),
        scratch_types=[
            pltpu.VMEM((K,), jnp.int32),
            pltpu.VMEM((K,), jnp.int32),
            pltpu.VMEM((K,), jnp.int32),
            pltpu.VMEM((K, DEG_LANES), jnp.float32),
            pltpu.VMEM_SHARED((NPAD, DEG_LANES), jnp.float32),
            pltpu.SemaphoreType.DMA,
            pltpu.SemaphoreType.DMA,
            pltpu.SemaphoreType.DMA,
            pltpu.SemaphoreType.DMA,
            pltpu.SemaphoreType.DMA,
            pltpu.SemaphoreType.DMA,
        ],
    )
    return kern(ei3, zeros_rows)


def _make_agg(D):
    def body(y_hbm, ei3_hbm, out0_hbm, out1_hbm,
             sidx_all, didx0, didx1, rows0, rows1,
             acc, isem, gsem0, gsem1, dsem0, dsem1, ssem0, ssem1):
        src2_hbm = ei3_hbm.at[0]
        dst2_hbm = ei3_hbm.at[1]
        c = lax.axis_index("c")
        s = lax.axis_index("s")
        wid = s * 2 + c
        row0 = s * ROWS_PER_TILE
        c0 = wid * _CPT
        i1 = pltpu.async_copy(src2_hbm.at[pl.ds(c0, _CPT)], sidx_all, isem)
        pltpu.sync_copy(y_hbm.at[pl.ds(row0, ROWS_PER_TILE)],
                        acc.at[pl.ds(row0, ROWS_PER_TILE)])
        plsc.subcore_barrier()
        i1.wait()

        slots = [(didx0, dsem0, gsem0, ssem0, rows0),
                 (didx1, dsem1, gsem1, ssem1, rows1)]
        for b, (didx, dsem, gsem, ssem, rows) in enumerate(slots):
            pltpu.async_copy(dst2_hbm.at[c0 + b], didx, dsem)
            pltpu.async_copy(y_hbm.at[sidx_all.at[b]], rows, gsem)

        def ring(t, carry):
            j0 = c0 + 2 * t
            for b, (didx, dsem, gsem, ssem, rows) in enumerate(slots):
                pltpu.make_async_copy(y_hbm.at[sidx_all.at[0]], rows,
                                      gsem).wait()
                pltpu.make_async_copy(dst2_hbm.at[j0 + b], didx, dsem).wait()
                pltpu.async_copy(rows, acc.at[didx], ssem, add=True)

            @pl.when(t < _CPT // 2 - 1)
            def _():
                for b, (didx, dsem, gsem, ssem, rows) in enumerate(slots):
                    pltpu.make_async_copy(rows, acc.at[didx], ssem).wait()
                    pltpu.async_copy(dst2_hbm.at[j0 + b + 2], didx, dsem)
                    pltpu.async_copy(y_hbm.at[sidx_all.at[2 * t + b + 2]],
                                     rows, gsem)

            return carry

        lax.fori_loop(0, _CPT // 2, ring, 0)
        for b, (didx, dsem, gsem, ssem, rows) in enumerate(slots):
            pltpu.make_async_copy(rows, acc.at[didx], ssem).wait()

        @pl.when(wid < _REM)
        def _():
            ct = _CPT * NW + wid
            pltpu.sync_copy(src2_hbm.at[ct], didx0)
            pltpu.sync_copy(dst2_hbm.at[ct], didx1)
            pltpu.async_copy(y_hbm.at[didx0], rows0, gsem0).wait()
            pltpu.sync_copy(rows0, acc.at[didx1], add=True)

        plsc.subcore_barrier()

        @pl.when(c == 0)
        def _():
            pltpu.sync_copy(acc.at[pl.ds(row0, ROWS_PER_TILE)],
                            out0_hbm.at[pl.ds(row0, ROWS_PER_TILE)])

        @pl.when(c == 1)
        def _():
            pltpu.sync_copy(acc.at[pl.ds(row0, ROWS_PER_TILE)],
                            out1_hbm.at[pl.ds(row0, ROWS_PER_TILE)])

    kern = pl.kernel(
        body,
        out_type=(jax.ShapeDtypeStruct((NPAD, D), jnp.float32),
                  jax.ShapeDtypeStruct((NPAD, D), jnp.float32)),
        mesh=_mesh(),
        compiler_params=pltpu.CompilerParams(use_tc_tiling_on_sc=False),
        scratch_types=[
            pltpu.VMEM((_CPT, K), jnp.int32),
            pltpu.VMEM((K,), jnp.int32),
            pltpu.VMEM((K,), jnp.int32),
            pltpu.VMEM((K, D), jnp.float32),
            pltpu.VMEM((K, D), jnp.float32),
            pltpu.VMEM_SHARED((NPAD, D), jnp.float32),
            pltpu.SemaphoreType.DMA,
            pltpu.SemaphoreType.DMA,
            pltpu.SemaphoreType.DMA,
            pltpu.SemaphoreType.DMA,
            pltpu.SemaphoreType.DMA,
            pltpu.SemaphoreType.DMA,
            pltpu.SemaphoreType.DMA,
        ],
    )
    return kern


_RB = 1280


def _prep1_body(x_ref, w_ref, d0_ref, d1_ref, y_ref, dinv_ref):
    deg = d0_ref[:, 0:1] + d1_ref[:, 0:1] + 1.0
    dinv = lax.rsqrt(deg)
    xw = jnp.dot(x_ref[...], w_ref[...], preferred_element_type=jnp.float32)
    y_ref[...] = xw * dinv
    dinv_ref[...] = dinv


def _prep1(x_pad, w1, d0, d1):
    grid = NPAD // _RB
    return pl.pallas_call(
        _prep1_body,
        grid=(grid,),
        in_specs=[
            pl.BlockSpec((_RB, 128), lambda i: (i, 0)),
            pl.BlockSpec((128, 128), lambda i: (0, 0)),
            pl.BlockSpec((_RB, DEG_LANES), lambda i: (i, 0)),
            pl.BlockSpec((_RB, DEG_LANES), lambda i: (i, 0)),
        ],
        out_specs=[
            pl.BlockSpec((_RB, 128), lambda i: (i, 0)),
            pl.BlockSpec((_RB, 1), lambda i: (i, 0)),
        ],
        out_shape=[
            jax.ShapeDtypeStruct((NPAD, 128), jnp.float32),
            jax.ShapeDtypeStruct((NPAD, 1), jnp.float32),
        ],
    )(x_pad, w1, d0, d1)


def _mid_body(p0_ref, p1_ref, y_ref, dinv_ref, b_ref, w_ref, y2_ref):
    dinv = dinv_ref[...]
    h = dinv * (p0_ref[...] + p1_ref[...] - y_ref[...]) + b_ref[...]
    h = jnp.maximum(h, 0.0)
    y2_ref[...] = jnp.dot(h, w_ref[...], preferred_element_type=jnp.float32) * dinv


def _mid(p0, p1, y1, dinv, b1, w2):
    grid = NPAD // _RB
    return pl.pallas_call(
        _mid_body,
        grid=(grid,),
        in_specs=[
            pl.BlockSpec((_RB, 128), lambda i: (i, 0)),
            pl.BlockSpec((_RB, 128), lambda i: (i, 0)),
            pl.BlockSpec((_RB, 128), lambda i: (i, 0)),
            pl.BlockSpec((_RB, 1), lambda i: (i, 0)),
            pl.BlockSpec((1, 128), lambda i: (0, 0)),
            pl.BlockSpec((128, 64), lambda i: (0, 0)),
        ],
        out_specs=pl.BlockSpec((_RB, 64), lambda i: (i, 0)),
        out_shape=jax.ShapeDtypeStruct((NPAD, 64), jnp.float32),
    )(p0, p1, y1, dinv, b1, w2)


_RF = 1000


def _final_body(p0_ref, p1_ref, y_ref, dinv_ref, b_ref, o_ref):
    t = dinv_ref[...] * (p0_ref[...] + p1_ref[...] - y_ref[...]) + b_ref[...]
    m = jnp.max(t, axis=1, keepdims=True)
    e = t - m
    lse = jnp.log(jnp.sum(jnp.exp(e), axis=1, keepdims=True))
    o_ref[...] = e - lse


def _final(p0, p1, y2, dinv, b2):
    grid = N_NODES // _RF
    return pl.pallas_call(
        _final_body,
        grid=(grid,),
        in_specs=[
            pl.BlockSpec((_RF, 64), lambda i: (i, 0)),
            pl.BlockSpec((_RF, 64), lambda i: (i, 0)),
            pl.BlockSpec((_RF, 64), lambda i: (i, 0)),
            pl.BlockSpec((_RF, 1), lambda i: (i, 0)),
            pl.BlockSpec((1, 64), lambda i: (0, 0)),
        ],
        out_specs=pl.BlockSpec((_RF, 64), lambda i: (i, 0)),
        out_shape=jax.ShapeDtypeStruct((N_NODES, 64), jnp.float32),
    )(p0, p1, y2, dinv, b2)


@jax.jit
def kernel(x, edge_index, W1, b1, W2, b2):
    ei3 = edge_index.astype(jnp.int32).reshape(2, NCHUNKS, K)
    x_pad = jnp.pad(x, ((0, NPAD - N_NODES), (0, 0)))
    zeros_rows = jnp.zeros((NPAD, DEG_LANES), jnp.float32)

    d0, d1 = _degree_partials(ei3, zeros_rows)
    y1, dinv = _prep1(x_pad, W1, d0, d1)

    agg128 = _make_agg(128)
    p10, p11 = agg128(y1, ei3)
    y2 = _mid(p10, p11, y1, dinv, b1.reshape(1, 128), W2)

    agg64 = _make_agg(64)
    p20, p21 = agg64(y2, ei3)
    return _final(p20, p21, y2, dinv, b2.reshape(1, 64))


# speedup vs baseline: 1.3301x; 1.1877x over previous
import functools

import jax
import jax.numpy as jnp
from jax import lax
from jax.experimental import pallas as pl
from jax.experimental.pallas import tpu as pltpu
from jax.experimental.pallas import tpu_sc as plsc

N_NODES = 10000
NPAD = 10240
ROWS_PER_TILE = NPAD // 16
E = 320000
K = 128
NCHUNKS = E // K
NW = 32
DEG_LANES = 16


def _mesh():
    return plsc.VectorSubcoreMesh(core_axis_name="c", subcore_axis_name="s")


_CPT = NCHUNKS // NW
_REM = NCHUNKS - _CPT * NW
_NT = _CPT // 3


def _deg_body(ei3_hbm, zeros_hbm, out0_hbm, out1_hbm, didx0, didx1, didx2,
              obuf, acc, dsem0, dsem1, dsem2, ssem0, ssem1, ssem2):
    dst2_hbm = ei3_hbm.at[1]
    c = lax.axis_index("c")
    s = lax.axis_index("s")
    wid = s * 2 + c
    c0 = wid * _CPT

    def fill(i, carry):
        obuf[i, :] = jnp.full((16,), 1.0, jnp.float32)
        return carry

    lax.fori_loop(0, K, fill, 0)
    row0 = s * ROWS_PER_TILE
    pltpu.sync_copy(zeros_hbm.at[pl.ds(row0, ROWS_PER_TILE)],
                    acc.at[pl.ds(row0, ROWS_PER_TILE)])
    plsc.subcore_barrier()

    pltpu.async_copy(dst2_hbm.at[c0], didx0, dsem0)
    pltpu.async_copy(dst2_hbm.at[c0 + 1], didx1, dsem1)
    pltpu.async_copy(dst2_hbm.at[c0 + 2], didx2, dsem2)

    slots = [(didx0, dsem0, ssem0), (didx1, dsem1, ssem1),
             (didx2, dsem2, ssem2)]

    def ring(t, carry):
        j0 = c0 + 3 * t
        for b, (didx, dsem, ssem) in enumerate(slots):
            pltpu.make_async_copy(dst2_hbm.at[j0 + b], didx, dsem).wait()
            pltpu.async_copy(obuf, acc.at[didx], ssem, add=True)

        @pl.when(t < _NT - 1)
        def _():
            for b, (didx, dsem, ssem) in enumerate(slots):
                pltpu.make_async_copy(obuf, acc.at[didx], ssem).wait()
                pltpu.async_copy(dst2_hbm.at[j0 + b + 3], didx, dsem)

        return carry

    lax.fori_loop(0, _NT, ring, 0)
    pltpu.make_async_copy(obuf, acc.at[didx0], ssem0).wait()
    pltpu.make_async_copy(obuf, acc.at[didx1], ssem1).wait()
    pltpu.make_async_copy(obuf, acc.at[didx2], ssem2).wait()

    @pl.when(wid < _REM)
    def _():
        ct = _CPT * NW + wid
        pltpu.sync_copy(dst2_hbm.at[ct], didx0)
        pltpu.sync_copy(obuf, acc.at[didx0], add=True)

    plsc.subcore_barrier()

    @pl.when(c == 0)
    def _():
        pltpu.sync_copy(acc.at[pl.ds(row0, ROWS_PER_TILE)],
                        out0_hbm.at[pl.ds(row0, ROWS_PER_TILE)])

    @pl.when(c == 1)
    def _():
        pltpu.sync_copy(acc.at[pl.ds(row0, ROWS_PER_TILE)],
                        out1_hbm.at[pl.ds(row0, ROWS_PER_TILE)])


def _degree_partials(ei3, zeros_rows):
    kern = pl.kernel(
        _deg_body,
        out_type=(jax.ShapeDtypeStruct((NPAD, DEG_LANES), jnp.float32),
                  jax.ShapeDtypeStruct((NPAD, DEG_LANES), jnp.float32)),
        mesh=_mesh(),
        compiler_params=pltpu.CompilerParams(use_tc_tiling_on_sc=False),
        scratch_types=[
            pltpu.VMEM((K,), jnp.int32),
            pltpu.VMEM((K,), jnp.int32),
            pltpu.VMEM((K,), jnp.int32),
            pltpu.VMEM((K, DEG_LANES), jnp.float32),
            pltpu.VMEM_SHARED((NPAD, DEG_LANES), jnp.float32),
            pltpu.SemaphoreType.DMA,
            pltpu.SemaphoreType.DMA,
            pltpu.SemaphoreType.DMA,
            pltpu.SemaphoreType.DMA,
            pltpu.SemaphoreType.DMA,
            pltpu.SemaphoreType.DMA,
        ],
    )
    return kern(ei3, zeros_rows)


def _make_agg(D):
    def body(y_hbm, ei3_hbm, out0_hbm, out1_hbm,
             sidx_all, didx0, didx1, rows0, rows1,
             acc, isem, gsem0, gsem1, dsem0, dsem1, ssem0, ssem1):
        src2_hbm = ei3_hbm.at[0]
        dst2_hbm = ei3_hbm.at[1]
        c = lax.axis_index("c")
        s = lax.axis_index("s")
        wid = s * 2 + c
        row0 = s * ROWS_PER_TILE
        c0 = wid * _CPT
        i1 = pltpu.async_copy(src2_hbm.at[pl.ds(c0, _CPT)], sidx_all, isem)
        pltpu.sync_copy(y_hbm.at[pl.ds(row0, ROWS_PER_TILE)],
                        acc.at[pl.ds(row0, ROWS_PER_TILE)])
        plsc.subcore_barrier()
        i1.wait()

        pltpu.async_copy(dst2_hbm.at[c0], didx0, dsem0)
        pltpu.async_copy(dst2_hbm.at[c0 + 1], didx1, dsem1)
        pltpu.async_copy(y_hbm.at[sidx_all.at[0]], rows0, gsem0)

        def pair(p, carry):
            ja = 2 * p
            pltpu.async_copy(y_hbm.at[sidx_all.at[ja + 1]], rows1, gsem1)
            pltpu.make_async_copy(y_hbm.at[sidx_all.at[ja]], rows0, gsem0).wait()
            pltpu.make_async_copy(dst2_hbm.at[c0], didx0, dsem0).wait()
            pltpu.sync_copy(rows0, acc.at[didx0], add=True)

            @pl.when(p < _CPT // 2 - 1)
            def _():
                pltpu.async_copy(y_hbm.at[sidx_all.at[ja + 2]], rows0, gsem0)
                pltpu.async_copy(dst2_hbm.at[c0 + ja + 2], didx0, dsem0)

            pltpu.make_async_copy(y_hbm.at[sidx_all.at[ja + 1]], rows1,
                                  gsem1).wait()
            pltpu.make_async_copy(dst2_hbm.at[c0], didx1, dsem1).wait()
            pltpu.sync_copy(rows1, acc.at[didx1], add=True)

            @pl.when(p < _CPT // 2 - 1)
            def _():
                pltpu.async_copy(dst2_hbm.at[c0 + ja + 3], didx1, dsem1)

            return carry

        lax.fori_loop(0, _CPT // 2, pair, 0)

        @pl.when(wid < _REM)
        def _():
            ct = _CPT * NW + wid
            pltpu.sync_copy(src2_hbm.at[ct], didx0)
            pltpu.sync_copy(dst2_hbm.at[ct], didx1)
            pltpu.async_copy(y_hbm.at[didx0], rows0, gsem0).wait()
            pltpu.sync_copy(rows0, acc.at[didx1], add=True)

        plsc.subcore_barrier()

        @pl.when(c == 0)
        def _():
            pltpu.sync_copy(acc.at[pl.ds(row0, ROWS_PER_TILE)],
                            out0_hbm.at[pl.ds(row0, ROWS_PER_TILE)])

        @pl.when(c == 1)
        def _():
            pltpu.sync_copy(acc.at[pl.ds(row0, ROWS_PER_TILE)],
                            out1_hbm.at[pl.ds(row0, ROWS_PER_TILE)])

    kern = pl.kernel(
        body,
        out_type=(jax.ShapeDtypeStruct((NPAD, D), jnp.float32),
                  jax.ShapeDtypeStruct((NPAD, D), jnp.float32)),
        mesh=_mesh(),
        compiler_params=pltpu.CompilerParams(use_tc_tiling_on_sc=False),
        scratch_types=[
            pltpu.VMEM((_CPT, K), jnp.int32),
            pltpu.VMEM((K,), jnp.int32),
            pltpu.VMEM((K,), jnp.int32),
            pltpu.VMEM((K, D), jnp.float32),
            pltpu.VMEM((K, D), jnp.float32),
            pltpu.VMEM_SHARED((NPAD, D), jnp.float32),
            pltpu.SemaphoreType.DMA,
            pltpu.SemaphoreType.DMA,
            pltpu.SemaphoreType.DMA,
            pltpu.SemaphoreType.DMA,
            pltpu.SemaphoreType.DMA,
            pltpu.SemaphoreType.DMA,
            pltpu.SemaphoreType.DMA,
        ],
    )
    return kern


_RB = 1280


def _prep1_body(x_ref, w_ref, d0_ref, d1_ref, y_ref, dinv_ref):
    deg = d0_ref[:, 0:1] + d1_ref[:, 0:1] + 1.0
    dinv = lax.rsqrt(deg)
    xw = jnp.dot(x_ref[...], w_ref[...], preferred_element_type=jnp.float32)
    y_ref[...] = xw * dinv
    dinv_ref[...] = dinv


def _prep1(x_pad, w1, d0, d1):
    grid = NPAD // _RB
    return pl.pallas_call(
        _prep1_body,
        grid=(grid,),
        in_specs=[
            pl.BlockSpec((_RB, 128), lambda i: (i, 0)),
            pl.BlockSpec((128, 128), lambda i: (0, 0)),
            pl.BlockSpec((_RB, DEG_LANES), lambda i: (i, 0)),
            pl.BlockSpec((_RB, DEG_LANES), lambda i: (i, 0)),
        ],
        out_specs=[
            pl.BlockSpec((_RB, 128), lambda i: (i, 0)),
            pl.BlockSpec((_RB, 1), lambda i: (i, 0)),
        ],
        out_shape=[
            jax.ShapeDtypeStruct((NPAD, 128), jnp.float32),
            jax.ShapeDtypeStruct((NPAD, 1), jnp.float32),
        ],
    )(x_pad, w1, d0, d1)


def _mid_body(p0_ref, p1_ref, y_ref, dinv_ref, b_ref, w_ref, y2_ref):
    dinv = dinv_ref[...]
    h = dinv * (p0_ref[...] + p1_ref[...] - y_ref[...]) + b_ref[...]
    h = jnp.maximum(h, 0.0)
    y2_ref[...] = jnp.dot(h, w_ref[...], preferred_element_type=jnp.float32) * dinv


def _mid(p0, p1, y1, dinv, b1, w2):
    grid = NPAD // _RB
    return pl.pallas_call(
        _mid_body,
        grid=(grid,),
        in_specs=[
            pl.BlockSpec((_RB, 128), lambda i: (i, 0)),
            pl.BlockSpec((_RB, 128), lambda i: (i, 0)),
            pl.BlockSpec((_RB, 128), lambda i: (i, 0)),
            pl.BlockSpec((_RB, 1), lambda i: (i, 0)),
            pl.BlockSpec((1, 128), lambda i: (0, 0)),
            pl.BlockSpec((128, 64), lambda i: (0, 0)),
        ],
        out_specs=pl.BlockSpec((_RB, 64), lambda i: (i, 0)),
        out_shape=jax.ShapeDtypeStruct((NPAD, 64), jnp.float32),
    )(p0, p1, y1, dinv, b1, w2)


_RF = 1000


def _final_body(p0_ref, p1_ref, y_ref, dinv_ref, b_ref, o_ref):
    t = dinv_ref[...] * (p0_ref[...] + p1_ref[...] - y_ref[...]) + b_ref[...]
    m = jnp.max(t, axis=1, keepdims=True)
    e = t - m
    lse = jnp.log(jnp.sum(jnp.exp(e), axis=1, keepdims=True))
    o_ref[...] = e - lse


def _final(p0, p1, y2, dinv, b2):
    grid = N_NODES // _RF
    return pl.pallas_call(
        _final_body,
        grid=(grid,),
        in_specs=[
            pl.BlockSpec((_RF, 64), lambda i: (i, 0)),
            pl.BlockSpec((_RF, 64), lambda i: (i, 0)),
            pl.BlockSpec((_RF, 64), lambda i: (i, 0)),
            pl.BlockSpec((_RF, 1), lambda i: (i, 0)),
            pl.BlockSpec((1, 64), lambda i: (0, 0)),
        ],
        out_specs=pl.BlockSpec((_RF, 64), lambda i: (i, 0)),
        out_shape=jax.ShapeDtypeStruct((N_NODES, 64), jnp.float32),
    )(p0, p1, y2, dinv, b2)


@jax.jit
def kernel(x, edge_index, W1, b1, W2, b2):
    ei3 = edge_index.astype(jnp.int32).reshape(2, NCHUNKS, K)
    x_pad = jnp.pad(x, ((0, NPAD - N_NODES), (0, 0)))
    zeros_rows = jnp.zeros((NPAD, DEG_LANES), jnp.float32)

    d0, d1 = _degree_partials(ei3, zeros_rows)
    y1, dinv = _prep1(x_pad, W1, d0, d1)

    agg128 = _make_agg(128)
    p10, p11 = agg128(y1, ei3)
    y2 = _mid(p10, p11, y1, dinv, b1.reshape(1, 128), W2)

    agg64 = _make_agg(64)
    p20, p21 = agg64(y2, ei3)
    return _final(p20, p21, y2, dinv, b2.reshape(1, 64))


# speedup vs baseline: 1.3667x; 1.0275x over previous
import functools

import jax
import jax.numpy as jnp
from jax import lax
from jax.experimental import pallas as pl
from jax.experimental.pallas import tpu as pltpu
from jax.experimental.pallas import tpu_sc as plsc

N_NODES = 10000
NPAD = 10240
ROWS_PER_TILE = NPAD // 16
E = 320000
K = 128
NCHUNKS = E // K
NW = 32
DEG_LANES = 16


def _mesh():
    return plsc.VectorSubcoreMesh(core_axis_name="c", subcore_axis_name="s")


_CPT = NCHUNKS // NW
_REM = NCHUNKS - _CPT * NW
_NT = _CPT // 3


def _deg_body(ei_hbm, zeros_hbm, out_hbm, didx0, didx1, didx2,
              obuf, acc, dsem0, dsem1, dsem2, ssem0, ssem1, ssem2):
    dst_hbm = ei_hbm.at[1]
    c = lax.axis_index("c")
    s = lax.axis_index("s")
    wid = s * 2 + c
    c0 = wid * _CPT

    def fill(i, carry):
        obuf[i, :] = jnp.full((16,), 1.0, jnp.float32)
        return carry

    lax.fori_loop(0, K, fill, 0)
    row0 = s * ROWS_PER_TILE
    pltpu.sync_copy(zeros_hbm.at[pl.ds(row0, ROWS_PER_TILE)],
                    acc.at[pl.ds(row0, ROWS_PER_TILE)])
    plsc.subcore_barrier()

    pltpu.async_copy(dst_hbm.at[pl.ds(c0 * K, K)], didx0, dsem0)
    pltpu.async_copy(dst_hbm.at[pl.ds((c0 + 1) * K, K)], didx1, dsem1)
    pltpu.async_copy(dst_hbm.at[pl.ds((c0 + 2) * K, K)], didx2, dsem2)

    slots = [(didx0, dsem0, ssem0), (didx1, dsem1, ssem1),
             (didx2, dsem2, ssem2)]

    def ring(t, carry):
        j0 = c0 + 3 * t
        for b, (didx, dsem, ssem) in enumerate(slots):
            pltpu.make_async_copy(dst_hbm.at[pl.ds((j0 + b) * K, K)], didx,
                                  dsem).wait()
            pltpu.async_copy(obuf, acc.at[didx], ssem, add=True)

        @pl.when(t < _NT - 1)
        def _():
            for b, (didx, dsem, ssem) in enumerate(slots):
                pltpu.make_async_copy(obuf, acc.at[didx], ssem).wait()
                pltpu.async_copy(dst_hbm.at[pl.ds((j0 + b + 3) * K, K)],
                                 didx, dsem)

        return carry

    lax.fori_loop(0, _NT, ring, 0)
    pltpu.make_async_copy(obuf, acc.at[didx0], ssem0).wait()
    pltpu.make_async_copy(obuf, acc.at[didx1], ssem1).wait()
    pltpu.make_async_copy(obuf, acc.at[didx2], ssem2).wait()

    @pl.when(wid < _REM)
    def _():
        ct = _CPT * NW + wid
        pltpu.sync_copy(dst_hbm.at[pl.ds(ct * K, K)], didx0)
        pltpu.sync_copy(obuf, acc.at[didx0], add=True)

    plsc.subcore_barrier()
    pltpu.sync_copy(acc.at[pl.ds(row0, ROWS_PER_TILE)],
                    out_hbm.at[pl.ds(row0, ROWS_PER_TILE),
                               pl.ds(DEG_LANES * c, DEG_LANES)])


def _degree_partials(ei, zeros_rows):
    kern = pl.kernel(
        _deg_body,
        out_type=jax.ShapeDtypeStruct((NPAD, 2 * DEG_LANES), jnp.float32),
        mesh=_mesh(),
        compiler_params=pltpu.CompilerParams(use_tc_tiling_on_sc=False),
        scratch_types=[
            pltpu.VMEM((K,), jnp.int32),
            pltpu.VMEM((K,), jnp.int32),
            pltpu.VMEM((K,), jnp.int32),
            pltpu.VMEM((K, DEG_LANES), jnp.float32),
            pltpu.VMEM_SHARED((NPAD, DEG_LANES), jnp.float32),
            pltpu.SemaphoreType.DMA,
            pltpu.SemaphoreType.DMA,
            pltpu.SemaphoreType.DMA,
            pltpu.SemaphoreType.DMA,
            pltpu.SemaphoreType.DMA,
            pltpu.SemaphoreType.DMA,
        ],
    )
    return kern(ei, zeros_rows)


def _make_agg(D, packed):
    def body(y_hbm, ei_hbm, *outs_and_scratch):
        if packed:
            (out_hbm, sidx_all, didx0, didx1, rows0, rows1,
             acc, isem, gsem0, gsem1, dsem0, dsem1, ssem0, ssem1) = \
                outs_and_scratch
        else:
            (out0_hbm, out1_hbm, sidx_all, didx0, didx1, rows0, rows1,
             acc, isem, gsem0, gsem1, dsem0, dsem1, ssem0, ssem1) = \
                outs_and_scratch
        src_hbm = ei_hbm.at[0]
        dst_hbm = ei_hbm.at[1]
        c = lax.axis_index("c")
        s = lax.axis_index("s")
        wid = s * 2 + c
        row0 = s * ROWS_PER_TILE
        c0 = wid * _CPT
        i1 = pltpu.async_copy(src_hbm.at[pl.ds(c0 * K, _CPT * K)], sidx_all,
                              isem)
        pltpu.sync_copy(y_hbm.at[pl.ds(row0, ROWS_PER_TILE)],
                        acc.at[pl.ds(row0, ROWS_PER_TILE)])
        plsc.subcore_barrier()
        i1.wait()

        def sidx(j):
            return sidx_all.at[pl.ds(j * K, K)]

        pltpu.async_copy(dst_hbm.at[pl.ds(c0 * K, K)], didx0, dsem0)
        pltpu.async_copy(dst_hbm.at[pl.ds((c0 + 1) * K, K)], didx1, dsem1)
        pltpu.async_copy(y_hbm.at[sidx(0)], rows0, gsem0)

        def pair(p, carry):
            ja = 2 * p
            pltpu.async_copy(y_hbm.at[sidx(ja + 1)], rows1, gsem1)
            pltpu.make_async_copy(y_hbm.at[sidx(ja)], rows0, gsem0).wait()
            pltpu.make_async_copy(dst_hbm.at[pl.ds(c0 * K, K)], didx0,
                                  dsem0).wait()
            pltpu.sync_copy(rows0, acc.at[didx0], add=True)

            @pl.when(p < _CPT // 2 - 1)
            def _():
                pltpu.async_copy(y_hbm.at[sidx(ja + 2)], rows0, gsem0)
                pltpu.async_copy(dst_hbm.at[pl.ds((c0 + ja + 2) * K, K)],
                                 didx0, dsem0)

            pltpu.make_async_copy(y_hbm.at[sidx(ja + 1)], rows1,
                                  gsem1).wait()
            pltpu.make_async_copy(dst_hbm.at[pl.ds(c0 * K, K)], didx1,
                                  dsem1).wait()
            pltpu.sync_copy(rows1, acc.at[didx1], add=True)

            @pl.when(p < _CPT // 2 - 1)
            def _():
                pltpu.async_copy(dst_hbm.at[pl.ds((c0 + ja + 3) * K, K)],
                                 didx1, dsem1)

            return carry

        lax.fori_loop(0, _CPT // 2, pair, 0)

        @pl.when(wid < _REM)
        def _():
            ct = _CPT * NW + wid
            pltpu.sync_copy(src_hbm.at[pl.ds(ct * K, K)], didx0)
            pltpu.sync_copy(dst_hbm.at[pl.ds(ct * K, K)], didx1)
            pltpu.async_copy(y_hbm.at[didx0], rows0, gsem0).wait()
            pltpu.sync_copy(rows0, acc.at[didx1], add=True)

        plsc.subcore_barrier()

        if packed:
            pltpu.sync_copy(acc.at[pl.ds(row0, ROWS_PER_TILE)],
                            out_hbm.at[pl.ds(row0, ROWS_PER_TILE),
                                       pl.ds(D * c, D)])
        else:
            @pl.when(c == 0)
            def _():
                pltpu.sync_copy(acc.at[pl.ds(row0, ROWS_PER_TILE)],
                                out0_hbm.at[pl.ds(row0, ROWS_PER_TILE)])

            @pl.when(c == 1)
            def _():
                pltpu.sync_copy(acc.at[pl.ds(row0, ROWS_PER_TILE)],
                                out1_hbm.at[pl.ds(row0, ROWS_PER_TILE)])

    if packed:
        out_type = jax.ShapeDtypeStruct((NPAD, 2 * D), jnp.float32)
    else:
        out_type = (jax.ShapeDtypeStruct((NPAD, D), jnp.float32),
                    jax.ShapeDtypeStruct((NPAD, D), jnp.float32))
    kern = pl.kernel(
        body,
        out_type=out_type,
        mesh=_mesh(),
        compiler_params=pltpu.CompilerParams(use_tc_tiling_on_sc=False),
        scratch_types=[
            pltpu.VMEM((_CPT * K,), jnp.int32),
            pltpu.VMEM((K,), jnp.int32),
            pltpu.VMEM((K,), jnp.int32),
            pltpu.VMEM((K, D), jnp.float32),
            pltpu.VMEM((K, D), jnp.float32),
            pltpu.VMEM_SHARED((NPAD, D), jnp.float32),
            pltpu.SemaphoreType.DMA,
            pltpu.SemaphoreType.DMA,
            pltpu.SemaphoreType.DMA,
            pltpu.SemaphoreType.DMA,
            pltpu.SemaphoreType.DMA,
            pltpu.SemaphoreType.DMA,
            pltpu.SemaphoreType.DMA,
        ],
    )
    return kern


_RB = 1280


def _mm1_body(x_ref, w_ref, xw_ref):
    xw_ref[...] = jnp.dot(x_ref[...], w_ref[...],
                          preferred_element_type=jnp.float32)


def _mm1(x_pad, w1):
    grid = NPAD // _RB
    return pl.pallas_call(
        _mm1_body,
        grid=(grid,),
        in_specs=[
            pl.BlockSpec((_RB, 128), lambda i: (i, 0)),
            pl.BlockSpec((128, 128), lambda i: (0, 0)),
        ],
        out_specs=pl.BlockSpec((_RB, 128), lambda i: (i, 0)),
        out_shape=jax.ShapeDtypeStruct((NPAD, 128), jnp.float32),
    )(x_pad, w1)


def _scale1_body(xw_ref, dp_ref, y_ref, dinv_ref):
    deg = dp_ref[:, 0:1] + dp_ref[:, DEG_LANES:DEG_LANES + 1] + 1.0
    dinv = lax.rsqrt(deg)
    y_ref[...] = xw_ref[...] * dinv
    dinv_ref[...] = dinv


def _scale1(xw, degp):
    grid = NPAD // _RB
    return pl.pallas_call(
        _scale1_body,
        grid=(grid,),
        in_specs=[
            pl.BlockSpec((_RB, 128), lambda i: (i, 0)),
            pl.BlockSpec((_RB, 2 * DEG_LANES), lambda i: (i, 0)),
        ],
        out_specs=[
            pl.BlockSpec((_RB, 128), lambda i: (i, 0)),
            pl.BlockSpec((_RB, 1), lambda i: (i, 0)),
        ],
        out_shape=[
            jax.ShapeDtypeStruct((NPAD, 128), jnp.float32),
            jax.ShapeDtypeStruct((NPAD, 1), jnp.float32),
        ],
    )(xw, degp)


def _mid_body(p0_ref, p1_ref, y_ref, dinv_ref, b_ref, w_ref, y2_ref):
    dinv = dinv_ref[...]
    h = dinv * (p0_ref[...] + p1_ref[...] - y_ref[...]) + b_ref[...]
    h = jnp.maximum(h, 0.0)
    y2_ref[...] = jnp.dot(h, w_ref[...], preferred_element_type=jnp.float32) * dinv


def _mid(p0, p1, y1, dinv, b1, w2):
    grid = NPAD // _RB
    return pl.pallas_call(
        _mid_body,
        grid=(grid,),
        in_specs=[
            pl.BlockSpec((_RB, 128), lambda i: (i, 0)),
            pl.BlockSpec((_RB, 128), lambda i: (i, 0)),
            pl.BlockSpec((_RB, 128), lambda i: (i, 0)),
            pl.BlockSpec((_RB, 1), lambda i: (i, 0)),
            pl.BlockSpec((1, 128), lambda i: (0, 0)),
            pl.BlockSpec((128, 64), lambda i: (0, 0)),
        ],
        out_specs=pl.BlockSpec((_RB, 64), lambda i: (i, 0)),
        out_shape=jax.ShapeDtypeStruct((NPAD, 64), jnp.float32),
    )(p0, p1, y1, dinv, b1, w2)


_RF = 1000


def _final_body(q_ref, y_ref, dinv_ref, b_ref, o_ref):
    q = q_ref[:, 0:64] + q_ref[:, 64:128]
    t = dinv_ref[...] * (q - y_ref[...]) + b_ref[...]
    m = jnp.max(t, axis=1, keepdims=True)
    e = t - m
    lse = jnp.log(jnp.sum(jnp.exp(e), axis=1, keepdims=True))
    o_ref[...] = e - lse


def _final(q, y2, dinv, b2):
    grid = N_NODES // _RF
    return pl.pallas_call(
        _final_body,
        grid=(grid,),
        in_specs=[
            pl.BlockSpec((_RF, 128), lambda i: (i, 0)),
            pl.BlockSpec((_RF, 64), lambda i: (i, 0)),
            pl.BlockSpec((_RF, 1), lambda i: (i, 0)),
            pl.BlockSpec((1, 64), lambda i: (0, 0)),
        ],
        out_specs=pl.BlockSpec((_RF, 64), lambda i: (i, 0)),
        out_shape=jax.ShapeDtypeStruct((N_NODES, 64), jnp.float32),
    )(q, y2, dinv, b2)


@jax.jit
def kernel(x, edge_index, W1, b1, W2, b2):
    ei = edge_index.astype(jnp.int32)
    x_pad = jnp.pad(x, ((0, NPAD - N_NODES), (0, 0)))
    zeros_rows = jnp.zeros((NPAD, DEG_LANES), jnp.float32)

    degp = _degree_partials(ei, zeros_rows)
    xw = _mm1(x_pad, W1)
    y1, dinv = _scale1(xw, degp)

    agg128 = _make_agg(128, packed=False)
    p10, p11 = agg128(y1, ei)
    y2 = _mid(p10, p11, y1, dinv, b1.reshape(1, 128), W2)

    agg64 = _make_agg(64, packed=True)
    q = agg64(y2, ei)
    return _final(q, y2, dinv, b2.reshape(1, 64))


# speedup vs baseline: 1.4299x; 1.0463x over previous
import functools

import jax
import jax.numpy as jnp
from jax import lax
from jax.experimental import pallas as pl
from jax.experimental.pallas import tpu as pltpu
from jax.experimental.pallas import tpu_sc as plsc

N_NODES = 10000
NPAD = 10240
ROWS_PER_TILE = NPAD // 16
E = 320000
K = 128
NCHUNKS = E // K
NW = 32
DEG_LANES = 16


def _mesh():
    return plsc.VectorSubcoreMesh(core_axis_name="c", subcore_axis_name="s")


_CPT = NCHUNKS // NW
_REM = NCHUNKS - _CPT * NW
_NT = _CPT // 3


def _deg_body(ei3_hbm, zeros_hbm, out_hbm, didx_all, didx_t,
              obuf, acc, isem, ssem, tsem):
    dst2_hbm = ei3_hbm.at[1]
    c = lax.axis_index("c")
    s = lax.axis_index("s")
    wid = s * 2 + c
    c0 = wid * _CPT
    i1 = pltpu.async_copy(dst2_hbm.at[pl.ds(c0, _CPT)], didx_all, isem)

    def fill(i, carry):
        obuf[i, :] = jnp.full((16,), 1.0, jnp.float32)
        return carry

    lax.fori_loop(0, K, fill, 0)
    row0 = s * ROWS_PER_TILE
    pltpu.sync_copy(zeros_hbm.at[pl.ds(row0, ROWS_PER_TILE)],
                    acc.at[pl.ds(row0, ROWS_PER_TILE)])
    plsc.subcore_barrier()
    i1.wait()

    def fire(j, carry):
        pltpu.async_copy(obuf, acc.at[didx_all.at[j]], ssem, add=True)
        return carry

    lax.fori_loop(0, _CPT, fire, 0)

    def drain(j, carry):
        pltpu.make_async_copy(obuf, acc.at[didx_all.at[0]], ssem).wait()
        return carry

    lax.fori_loop(0, _CPT, drain, 0)

    @pl.when(wid < _REM)
    def _():
        ct = _CPT * NW + wid
        pltpu.sync_copy(dst2_hbm.at[ct], didx_t)
        pltpu.async_copy(obuf, acc.at[didx_t], tsem, add=True).wait()

    plsc.subcore_barrier()
    pltpu.sync_copy(acc.at[pl.ds(row0, ROWS_PER_TILE)],
                    out_hbm.at[pl.ds(row0, ROWS_PER_TILE),
                               pl.ds(DEG_LANES * c, DEG_LANES)])


def _degree_partials(ei, zeros_rows):
    kern = pl.kernel(
        _deg_body,
        out_type=jax.ShapeDtypeStruct((NPAD, 2 * DEG_LANES), jnp.float32),
        mesh=_mesh(),
        compiler_params=pltpu.CompilerParams(use_tc_tiling_on_sc=False),
        scratch_types=[
            pltpu.VMEM((_CPT, K), jnp.int32),
            pltpu.VMEM((K,), jnp.int32),
            pltpu.VMEM((K, DEG_LANES), jnp.float32),
            pltpu.VMEM_SHARED((NPAD, DEG_LANES), jnp.float32),
            pltpu.SemaphoreType.DMA,
            pltpu.SemaphoreType.DMA,
            pltpu.SemaphoreType.DMA,
        ],
    )
    return kern(ei, zeros_rows)


def _make_agg(D, packed):
    def body(y_hbm, ei3_hbm, *outs_and_scratch):
        if packed:
            (out_hbm, sidx_all, didx_all, sidx0, sidx1, rows0, rows1,
             acc, isem, gsem0, gsem1, esem0, esem1, tsem) = outs_and_scratch
        else:
            (out0_hbm, out1_hbm, sidx_all, didx_all, sidx0, sidx1, rows0,
             rows1, acc, isem, gsem0, gsem1, esem0, esem1, tsem) = \
                outs_and_scratch
        src2_hbm = ei3_hbm.at[0]
        dst2_hbm = ei3_hbm.at[1]
        c = lax.axis_index("c")
        s = lax.axis_index("s")
        wid = s * 2 + c
        row0 = s * ROWS_PER_TILE
        c0 = wid * _CPT
        i1 = pltpu.async_copy(dst2_hbm.at[pl.ds(c0, _CPT)], didx_all, isem)
        if packed:
            i2 = pltpu.async_copy(src2_hbm.at[pl.ds(c0, _CPT)], sidx_all,
                                  isem)
        else:
            pltpu.async_copy(src2_hbm.at[c0], sidx0, esem0)
            pltpu.async_copy(src2_hbm.at[c0 + 1], sidx1, esem1)
        pltpu.sync_copy(y_hbm.at[pl.ds(row0, ROWS_PER_TILE)],
                        acc.at[pl.ds(row0, ROWS_PER_TILE)])
        plsc.subcore_barrier()
        i1.wait()
        if packed:
            i2.wait()

            def gidx(j):
                return sidx_all.at[j]
        else:
            pltpu.make_async_copy(src2_hbm.at[c0], sidx0, esem0).wait()
            pltpu.make_async_copy(src2_hbm.at[c0], sidx1, esem1).wait()

        if packed:
            pltpu.async_copy(y_hbm.at[gidx(0)], rows0, gsem0)
            pltpu.async_copy(y_hbm.at[gidx(1)], rows1, gsem1)

            def pair(p, carry):
                ja = 2 * p
                pltpu.make_async_copy(y_hbm.at[gidx(0)], rows0, gsem0).wait()
                pltpu.sync_copy(rows0, acc.at[didx_all.at[ja]], add=True)

                @pl.when(p < _CPT // 2 - 1)
                def _():
                    pltpu.async_copy(y_hbm.at[gidx(ja + 2)], rows0, gsem0)

                pltpu.make_async_copy(y_hbm.at[gidx(0)], rows1, gsem1).wait()
                pltpu.sync_copy(rows1, acc.at[didx_all.at[ja + 1]], add=True)

                @pl.when(p < _CPT // 2 - 1)
                def _():
                    pltpu.async_copy(y_hbm.at[gidx(ja + 3)], rows1, gsem1)

                return carry
        else:
            pltpu.async_copy(y_hbm.at[sidx0], rows0, gsem0)
            pltpu.async_copy(y_hbm.at[sidx1], rows1, gsem1)

            def pair(p, carry):
                ja = 2 * p
                pltpu.make_async_copy(y_hbm.at[sidx0], rows0, gsem0).wait()

                @pl.when(p < _CPT // 2 - 1)
                def _():
                    pltpu.async_copy(src2_hbm.at[c0 + ja + 2], sidx0, esem0)

                pltpu.sync_copy(rows0, acc.at[didx_all.at[ja]], add=True)

                @pl.when(p < _CPT // 2 - 1)
                def _():
                    pltpu.make_async_copy(src2_hbm.at[c0], sidx0,
                                          esem0).wait()
                    pltpu.async_copy(y_hbm.at[sidx0], rows0, gsem0)

                pltpu.make_async_copy(y_hbm.at[sidx1], rows1, gsem1).wait()

                @pl.when(p < _CPT // 2 - 1)
                def _():
                    pltpu.async_copy(src2_hbm.at[c0 + ja + 3], sidx1, esem1)

                pltpu.sync_copy(rows1, acc.at[didx_all.at[ja + 1]], add=True)

                @pl.when(p < _CPT // 2 - 1)
                def _():
                    pltpu.make_async_copy(src2_hbm.at[c0], sidx1,
                                          esem1).wait()
                    pltpu.async_copy(y_hbm.at[sidx1], rows1, gsem1)

                return carry

        lax.fori_loop(0, _CPT // 2, pair, 0)

        @pl.when(wid < _REM)
        def _():
            ct = _CPT * NW + wid
            pltpu.sync_copy(src2_hbm.at[ct], sidx0)
            pltpu.sync_copy(dst2_hbm.at[ct], sidx1)
            pltpu.async_copy(y_hbm.at[sidx0], rows0, tsem).wait()
            pltpu.sync_copy(rows0, acc.at[sidx1], add=True)

        plsc.subcore_barrier()

        if packed:
            pltpu.sync_copy(acc.at[pl.ds(row0, ROWS_PER_TILE)],
                            out_hbm.at[pl.ds(row0, ROWS_PER_TILE),
                                       pl.ds(D * c, D)])
        else:
            @pl.when(c == 0)
            def _():
                pltpu.sync_copy(acc.at[pl.ds(row0, ROWS_PER_TILE)],
                                out0_hbm.at[pl.ds(row0, ROWS_PER_TILE)])

            @pl.when(c == 1)
            def _():
                pltpu.sync_copy(acc.at[pl.ds(row0, ROWS_PER_TILE)],
                                out1_hbm.at[pl.ds(row0, ROWS_PER_TILE)])

    if packed:
        out_type = jax.ShapeDtypeStruct((NPAD, 2 * D), jnp.float32)
        scratch = [
            pltpu.VMEM((_CPT, K), jnp.int32),
            pltpu.VMEM((_CPT, K), jnp.int32),
            pltpu.VMEM((K,), jnp.int32),
            pltpu.VMEM((K,), jnp.int32),
            pltpu.VMEM((K, D), jnp.float32),
            pltpu.VMEM((K, D), jnp.float32),
            pltpu.VMEM_SHARED((NPAD, D), jnp.float32),
        ]
    else:
        out_type = (jax.ShapeDtypeStruct((NPAD, D), jnp.float32),
                    jax.ShapeDtypeStruct((NPAD, D), jnp.float32))
        scratch = [
            pltpu.VMEM((1, K), jnp.int32),
            pltpu.VMEM((_CPT, K), jnp.int32),
            pltpu.VMEM((K,), jnp.int32),
            pltpu.VMEM((K,), jnp.int32),
            pltpu.VMEM((K, D), jnp.float32),
            pltpu.VMEM((K, D), jnp.float32),
            pltpu.VMEM_SHARED((NPAD, D), jnp.float32),
        ]
    kern = pl.kernel(
        body,
        out_type=out_type,
        mesh=_mesh(),
        compiler_params=pltpu.CompilerParams(use_tc_tiling_on_sc=False),
        scratch_types=scratch + [
            pltpu.SemaphoreType.DMA,
            pltpu.SemaphoreType.DMA,
            pltpu.SemaphoreType.DMA,
            pltpu.SemaphoreType.DMA,
            pltpu.SemaphoreType.DMA,
            pltpu.SemaphoreType.DMA,
        ],
    )
    return kern


_RB = 1280


def _mm1_body(x_ref, w_ref, xw_ref):
    xw_ref[...] = jnp.dot(x_ref[...], w_ref[...],
                          preferred_element_type=jnp.float32)


def _mm1(x_pad, w1):
    grid = NPAD // _RB
    return pl.pallas_call(
        _mm1_body,
        grid=(grid,),
        in_specs=[
            pl.BlockSpec((_RB, 128), lambda i: (i, 0)),
            pl.BlockSpec((128, 128), lambda i: (0, 0)),
        ],
        out_specs=pl.BlockSpec((_RB, 128), lambda i: (i, 0)),
        out_shape=jax.ShapeDtypeStruct((NPAD, 128), jnp.float32),
    )(x_pad, w1)


def _dinv_of(dp):
    return lax.rsqrt(dp[:, 0:1] + dp[:, DEG_LANES:DEG_LANES + 1] + 1.0)


def _scale1_body(xw_ref, dp_ref, y_ref):
    y_ref[...] = xw_ref[...] * _dinv_of(dp_ref[...])


def _scale1(xw, degp):
    grid = NPAD // _RB
    return pl.pallas_call(
        _scale1_body,
        grid=(grid,),
        in_specs=[
            pl.BlockSpec((_RB, 128), lambda i: (i, 0)),
            pl.BlockSpec((_RB, 2 * DEG_LANES), lambda i: (i, 0)),
        ],
        out_specs=pl.BlockSpec((_RB, 128), lambda i: (i, 0)),
        out_shape=jax.ShapeDtypeStruct((NPAD, 128), jnp.float32),
    )(xw, degp)


def _mid_body(p0_ref, p1_ref, y_ref, dp_ref, b_ref, w_ref, y2_ref):
    dinv = _dinv_of(dp_ref[...])
    h = dinv * (p0_ref[...] + p1_ref[...] - y_ref[...]) + b_ref[...]
    h = jnp.maximum(h, 0.0)
    y2_ref[...] = jnp.dot(h, w_ref[...], preferred_element_type=jnp.float32) * dinv


def _mid(p0, p1, y1, degp, b1, w2):
    grid = NPAD // _RB
    return pl.pallas_call(
        _mid_body,
        grid=(grid,),
        in_specs=[
            pl.BlockSpec((_RB, 128), lambda i: (i, 0)),
            pl.BlockSpec((_RB, 128), lambda i: (i, 0)),
            pl.BlockSpec((_RB, 128), lambda i: (i, 0)),
            pl.BlockSpec((_RB, 2 * DEG_LANES), lambda i: (i, 0)),
            pl.BlockSpec((1, 128), lambda i: (0, 0)),
            pl.BlockSpec((128, 64), lambda i: (0, 0)),
        ],
        out_specs=pl.BlockSpec((_RB, 64), lambda i: (i, 0)),
        out_shape=jax.ShapeDtypeStruct((NPAD, 64), jnp.float32),
    )(p0, p1, y1, degp, b1, w2)


_RF = 1000


def _final_body(q_ref, y_ref, dp_ref, b_ref, o_ref):
    q = q_ref[:, 0:64] + q_ref[:, 64:128]
    t = _dinv_of(dp_ref[...]) * (q - y_ref[...]) + b_ref[...]
    m = jnp.max(t, axis=1, keepdims=True)
    e = t - m
    lse = jnp.log(jnp.sum(jnp.exp(e), axis=1, keepdims=True))
    o_ref[...] = e - lse


def _final(q, y2, degp, b2):
    grid = N_NODES // _RF
    return pl.pallas_call(
        _final_body,
        grid=(grid,),
        in_specs=[
            pl.BlockSpec((_RF, 128), lambda i: (i, 0)),
            pl.BlockSpec((_RF, 64), lambda i: (i, 0)),
            pl.BlockSpec((_RF, 2 * DEG_LANES), lambda i: (i, 0)),
            pl.BlockSpec((1, 64), lambda i: (0, 0)),
        ],
        out_specs=pl.BlockSpec((_RF, 64), lambda i: (i, 0)),
        out_shape=jax.ShapeDtypeStruct((N_NODES, 64), jnp.float32),
    )(q, y2, degp, b2)


@jax.jit
def kernel(x, edge_index, W1, b1, W2, b2):
    ei3 = edge_index.astype(jnp.int32).reshape(2, NCHUNKS, K)
    x_pad = jnp.pad(x, ((0, NPAD - N_NODES), (0, 0)))
    zeros_rows = jnp.zeros((NPAD, DEG_LANES), jnp.float32)

    degp = _degree_partials(ei3, zeros_rows)
    xw = _mm1(x_pad, W1)
    y1 = _scale1(xw, degp)

    agg128 = _make_agg(128, packed=False)
    p10, p11 = agg128(y1, ei3)
    y2 = _mid(p10, p11, y1, degp, b1.reshape(1, 128), W2)

    agg64 = _make_agg(64, packed=True)
    q = agg64(y2, ei3)
    return _final(q, y2, degp, b2.reshape(1, 64))


# speedup vs baseline: 1.4411x; 1.0078x over previous
import jax
import jax.numpy as jnp
from jax import lax
from jax.experimental import pallas as pl
from jax.experimental.pallas import tpu as pltpu
from jax.experimental.pallas import tpu_sc as plsc

N_NODES = 10000
NPAD = 10240
ROWS_PER_TILE = NPAD // 16
E = 320000
K = 128
NCHUNKS = E // K
NW = 32
DEG_LANES = 16


def _mesh():
    return plsc.VectorSubcoreMesh(core_axis_name="c", subcore_axis_name="s")


_CPT = NCHUNKS // NW
_REM = NCHUNKS - _CPT * NW
_NT = _CPT // 3


def _deg_body(ei3_hbm, zeros_hbm, out_hbm, didx_all, didx_t,
              obuf, acc, isem, ssem, tsem):
    dst2_hbm = ei3_hbm.at[1]
    c = lax.axis_index("c")
    s = lax.axis_index("s")
    wid = s * 2 + c
    c0 = wid * _CPT
    i1 = pltpu.async_copy(dst2_hbm.at[pl.ds(c0, _CPT)], didx_all, isem)

    def fill(i, carry):
        obuf[i, :] = jnp.full((16,), 1.0, jnp.float32)
        return carry

    lax.fori_loop(0, K, fill, 0)
    row0 = s * ROWS_PER_TILE
    pltpu.sync_copy(zeros_hbm.at[pl.ds(row0, ROWS_PER_TILE)],
                    acc.at[pl.ds(row0, ROWS_PER_TILE)])
    plsc.subcore_barrier()
    i1.wait()

    def fire(j, carry):
        pltpu.async_copy(obuf, acc.at[didx_all.at[j]], ssem, add=True)
        return carry

    lax.fori_loop(0, _CPT, fire, 0)

    def drain(j, carry):
        pltpu.make_async_copy(obuf, acc.at[didx_all.at[0]], ssem).wait()
        return carry

    lax.fori_loop(0, _CPT, drain, 0)

    @pl.when(wid < _REM)
    def _():
        ct = _CPT * NW + wid
        pltpu.sync_copy(dst2_hbm.at[ct], didx_t)
        pltpu.async_copy(obuf, acc.at[didx_t], tsem, add=True).wait()

    plsc.subcore_barrier()
    pltpu.sync_copy(acc.at[pl.ds(row0, ROWS_PER_TILE)],
                    out_hbm.at[pl.ds(row0, ROWS_PER_TILE),
                               pl.ds(DEG_LANES * c, DEG_LANES)])


def _degree_partials(ei, zeros_rows):
    kern = pl.kernel(
        _deg_body,
        out_type=jax.ShapeDtypeStruct((NPAD, 2 * DEG_LANES), jnp.float32),
        mesh=_mesh(),
        compiler_params=pltpu.CompilerParams(use_tc_tiling_on_sc=False),
        scratch_types=[
            pltpu.VMEM((_CPT, K), jnp.int32),
            pltpu.VMEM((K,), jnp.int32),
            pltpu.VMEM((K, DEG_LANES), jnp.float32),
            pltpu.VMEM_SHARED((NPAD, DEG_LANES), jnp.float32),
            pltpu.SemaphoreType.DMA,
            pltpu.SemaphoreType.DMA,
            pltpu.SemaphoreType.DMA,
        ],
    )
    return kern(ei, zeros_rows)


def _make_agg(D, packed):
    def body(y_hbm, ei3_hbm, *outs_and_scratch):
        if packed:
            (out_hbm, sidx_all, didx_all, sidx0, sidx1, rows0, rows1,
             acc, isem, gsem0, gsem1, esem0, esem1, tsem) = outs_and_scratch
        else:
            (out0_hbm, out1_hbm, sidx_all, didx_all, sidx0, sidx1, rows0,
             rows1, acc, isem, gsem0, gsem1, esem0, esem1, tsem) = \
                outs_and_scratch
        src2_hbm = ei3_hbm.at[0]
        dst2_hbm = ei3_hbm.at[1]
        c = lax.axis_index("c")
        s = lax.axis_index("s")
        wid = s * 2 + c
        row0 = s * ROWS_PER_TILE
        c0 = wid * _CPT
        i1 = pltpu.async_copy(dst2_hbm.at[pl.ds(c0, _CPT)], didx_all, isem)
        if packed:
            i2 = pltpu.async_copy(src2_hbm.at[pl.ds(c0, _CPT)], sidx_all,
                                  isem)
        else:
            pltpu.async_copy(src2_hbm.at[c0], sidx0, esem0)
            pltpu.async_copy(src2_hbm.at[c0 + 1], sidx1, esem1)
        @pl.when(c == 0)
        def _():
            pltpu.sync_copy(y_hbm.at[pl.ds(row0, ROWS_PER_TILE)],
                            acc.at[pl.ds(row0, ROWS_PER_TILE)])

        @pl.when(c == 1)
        def _():
            def zfill(i, carry):
                for jj in range(D // 16):
                    rows0[i, pl.ds(jj * 16, 16)] = jnp.zeros((16,),
                                                             jnp.float32)
                return carry

            lax.fori_loop(0, K, zfill, 0)
            for blk in range(ROWS_PER_TILE // K):
                pltpu.sync_copy(rows0,
                                acc.at[pl.ds(row0 + blk * K, K)])

        plsc.subcore_barrier()
        i1.wait()
        if packed:
            i2.wait()

            def gidx(j):
                return sidx_all.at[j]
        else:
            pltpu.make_async_copy(src2_hbm.at[c0], sidx0, esem0).wait()
            pltpu.make_async_copy(src2_hbm.at[c0], sidx1, esem1).wait()

        if packed:
            pltpu.async_copy(y_hbm.at[gidx(0)], rows0, gsem0)
            pltpu.async_copy(y_hbm.at[gidx(1)], rows1, gsem1)

            def pair(p, carry):
                ja = 2 * p
                pltpu.make_async_copy(y_hbm.at[gidx(0)], rows0, gsem0).wait()
                pltpu.sync_copy(rows0, acc.at[didx_all.at[ja]], add=True)

                @pl.when(p < _CPT // 2 - 1)
                def _():
                    pltpu.async_copy(y_hbm.at[gidx(ja + 2)], rows0, gsem0)

                pltpu.make_async_copy(y_hbm.at[gidx(0)], rows1, gsem1).wait()
                pltpu.sync_copy(rows1, acc.at[didx_all.at[ja + 1]], add=True)

                @pl.when(p < _CPT // 2 - 1)
                def _():
                    pltpu.async_copy(y_hbm.at[gidx(ja + 3)], rows1, gsem1)

                return carry
        else:
            pltpu.async_copy(y_hbm.at[sidx0], rows0, gsem0)
            pltpu.async_copy(y_hbm.at[sidx1], rows1, gsem1)

            def pair(p, carry):
                ja = 2 * p
                pltpu.make_async_copy(y_hbm.at[sidx0], rows0, gsem0).wait()

                @pl.when(p < _CPT // 2 - 1)
                def _():
                    pltpu.async_copy(src2_hbm.at[c0 + ja + 2], sidx0, esem0)

                pltpu.sync_copy(rows0, acc.at[didx_all.at[ja]], add=True)

                @pl.when(p < _CPT // 2 - 1)
                def _():
                    pltpu.make_async_copy(src2_hbm.at[c0], sidx0,
                                          esem0).wait()
                    pltpu.async_copy(y_hbm.at[sidx0], rows0, gsem0)

                pltpu.make_async_copy(y_hbm.at[sidx1], rows1, gsem1).wait()

                @pl.when(p < _CPT // 2 - 1)
                def _():
                    pltpu.async_copy(src2_hbm.at[c0 + ja + 3], sidx1, esem1)

                pltpu.sync_copy(rows1, acc.at[didx_all.at[ja + 1]], add=True)

                @pl.when(p < _CPT // 2 - 1)
                def _():
                    pltpu.make_async_copy(src2_hbm.at[c0], sidx1,
                                          esem1).wait()
                    pltpu.async_copy(y_hbm.at[sidx1], rows1, gsem1)

                return carry

        lax.fori_loop(0, _CPT // 2, pair, 0)

        @pl.when(wid < _REM)
        def _():
            ct = _CPT * NW + wid
            pltpu.sync_copy(src2_hbm.at[ct], sidx0)
            pltpu.sync_copy(dst2_hbm.at[ct], sidx1)
            pltpu.async_copy(y_hbm.at[sidx0], rows0, tsem).wait()
            pltpu.sync_copy(rows0, acc.at[sidx1], add=True)

        plsc.subcore_barrier()

        if packed:
            pltpu.sync_copy(acc.at[pl.ds(row0, ROWS_PER_TILE)],
                            out_hbm.at[pl.ds(row0, ROWS_PER_TILE),
                                       pl.ds(D * c, D)])
        else:
            @pl.when(c == 0)
            def _():
                pltpu.sync_copy(acc.at[pl.ds(row0, ROWS_PER_TILE)],
                                out0_hbm.at[pl.ds(row0, ROWS_PER_TILE)])

            @pl.when(c == 1)
            def _():
                pltpu.sync_copy(acc.at[pl.ds(row0, ROWS_PER_TILE)],
                                out1_hbm.at[pl.ds(row0, ROWS_PER_TILE)])

    if packed:
        out_type = jax.ShapeDtypeStruct((NPAD, 2 * D), jnp.float32)
        scratch = [
            pltpu.VMEM((_CPT, K), jnp.int32),
            pltpu.VMEM((_CPT, K), jnp.int32),
            pltpu.VMEM((K,), jnp.int32),
            pltpu.VMEM((K,), jnp.int32),
            pltpu.VMEM((K, D), jnp.float32),
            pltpu.VMEM((K, D), jnp.float32),
            pltpu.VMEM_SHARED((NPAD, D), jnp.float32),
        ]
    else:
        out_type = (jax.ShapeDtypeStruct((NPAD, D), jnp.float32),
                    jax.ShapeDtypeStruct((NPAD, D), jnp.float32))
        scratch = [
            pltpu.VMEM((1, K), jnp.int32),
            pltpu.VMEM((_CPT, K), jnp.int32),
            pltpu.VMEM((K,), jnp.int32),
            pltpu.VMEM((K,), jnp.int32),
            pltpu.VMEM((K, D), jnp.float32),
            pltpu.VMEM((K, D), jnp.float32),
            pltpu.VMEM_SHARED((NPAD, D), jnp.float32),
        ]
    kern = pl.kernel(
        body,
        out_type=out_type,
        mesh=_mesh(),
        compiler_params=pltpu.CompilerParams(use_tc_tiling_on_sc=False),
        scratch_types=scratch + [
            pltpu.SemaphoreType.DMA,
            pltpu.SemaphoreType.DMA,
            pltpu.SemaphoreType.DMA,
            pltpu.SemaphoreType.DMA,
            pltpu.SemaphoreType.DMA,
            pltpu.SemaphoreType.DMA,
        ],
    )
    return kern


_RB = 1280


def _mm1_body(x_ref, w_ref, xw_ref):
    xw_ref[...] = jnp.dot(x_ref[...], w_ref[...],
                          preferred_element_type=jnp.float32)


def _mm1(x_pad, w1):
    grid = NPAD // _RB
    return pl.pallas_call(
        _mm1_body,
        grid=(grid,),
        in_specs=[
            pl.BlockSpec((_RB, 128), lambda i: (i, 0)),
            pl.BlockSpec((128, 128), lambda i: (0, 0)),
        ],
        out_specs=pl.BlockSpec((_RB, 128), lambda i: (i, 0)),
        out_shape=jax.ShapeDtypeStruct((NPAD, 128), jnp.float32),
    )(x_pad, w1)


def _dinv_of(dp):
    return lax.rsqrt(dp[:, 0:1] + dp[:, DEG_LANES:DEG_LANES + 1] + 1.0)


def _scale1_body(xw_ref, dp_ref, y_ref):
    y_ref[...] = xw_ref[...] * _dinv_of(dp_ref[...])


def _scale1(xw, degp):
    grid = NPAD // _RB
    return pl.pallas_call(
        _scale1_body,
        grid=(grid,),
        in_specs=[
            pl.BlockSpec((_RB, 128), lambda i: (i, 0)),
            pl.BlockSpec((_RB, 2 * DEG_LANES), lambda i: (i, 0)),
        ],
        out_specs=pl.BlockSpec((_RB, 128), lambda i: (i, 0)),
        out_shape=jax.ShapeDtypeStruct((NPAD, 128), jnp.float32),
    )(xw, degp)


def _mid_body(p0_ref, p1_ref, dp_ref, b_ref, w_ref, y2_ref):
    dinv = _dinv_of(dp_ref[...])
    h = dinv * (p0_ref[...] + p1_ref[...]) + b_ref[...]
    h = jnp.maximum(h, 0.0)
    y2_ref[...] = jnp.dot(h, w_ref[...], preferred_element_type=jnp.float32) * dinv


def _mid(p0, p1, degp, b1, w2):
    grid = NPAD // _RB
    return pl.pallas_call(
        _mid_body,
        grid=(grid,),
        in_specs=[
            pl.BlockSpec((_RB, 128), lambda i: (i, 0)),
            pl.BlockSpec((_RB, 128), lambda i: (i, 0)),
            pl.BlockSpec((_RB, 2 * DEG_LANES), lambda i: (i, 0)),
            pl.BlockSpec((1, 128), lambda i: (0, 0)),
            pl.BlockSpec((128, 64), lambda i: (0, 0)),
        ],
        out_specs=pl.BlockSpec((_RB, 64), lambda i: (i, 0)),
        out_shape=jax.ShapeDtypeStruct((NPAD, 64), jnp.float32),
    )(p0, p1, degp, b1, w2)


_RF = 1000


def _final_body(q_ref, dp_ref, b_ref, o_ref):
    q = q_ref[:, 0:64] + q_ref[:, 64:128]
    t = _dinv_of(dp_ref[...]) * q + b_ref[...]
    m = jnp.max(t, axis=1, keepdims=True)
    e = t - m
    lse = jnp.log(jnp.sum(jnp.exp(e), axis=1, keepdims=True))
    o_ref[...] = e - lse


def _final(q, degp, b2):
    grid = N_NODES // _RF
    return pl.pallas_call(
        _final_body,
        grid=(grid,),
        in_specs=[
            pl.BlockSpec((_RF, 128), lambda i: (i, 0)),
            pl.BlockSpec((_RF, 2 * DEG_LANES), lambda i: (i, 0)),
            pl.BlockSpec((1, 64), lambda i: (0, 0)),
        ],
        out_specs=pl.BlockSpec((_RF, 64), lambda i: (i, 0)),
        out_shape=jax.ShapeDtypeStruct((N_NODES, 64), jnp.float32),
    )(q, degp, b2)


@jax.jit
def kernel(x, edge_index, W1, b1, W2, b2):
    ei3 = edge_index.astype(jnp.int32).reshape(2, NCHUNKS, K)
    x_pad = jnp.pad(x, ((0, NPAD - N_NODES), (0, 0)))
    zeros_rows = jnp.zeros((NPAD, DEG_LANES), jnp.float32)

    degp = _degree_partials(ei3, zeros_rows)
    xw = _mm1(x_pad, W1)
    y1 = _scale1(xw, degp)

    agg128 = _make_agg(128, packed=False)
    p10, p11 = agg128(y1, ei3)
    y2 = _mid(p10, p11, degp, b1.reshape(1, 128), W2)

    agg64 = _make_agg(64, packed=True)
    q = agg64(y2, ei3)
    return _final(q, degp, b2.reshape(1, 64))
